# Initial kernel scaffold; baseline (speedup 1.0000x reference)
import functools

import jax
import jax.numpy as jnp
from jax import lax
from jax.experimental import pallas as pl
from jax.experimental.pallas import tpu as pltpu
from jax.experimental.pallas import tpu_sc as plsc

N = 10000
E = 320000
D = 128
DP = 144
NC = 2
NS = 16
NW = NC * NS
EPT = E // NW
CH = 80
NCHUNKS = EPT // CH
NPT = N // NS
LANES = 16


def _prep1_body(x_ref, w_ref, asv_ref, adv_ref,
                hp_ref, as_ref, ad_ref, m_ref):
    h = jnp.dot(x_ref[...], w_ref[...], preferred_element_type=jnp.float32)
    a_s = jnp.dot(h, asv_ref[...], preferred_element_type=jnp.float32)
    a_d = jnp.dot(h, adv_ref[...], preferred_element_type=jnp.float32)
    cols = lax.broadcasted_iota(jnp.int32, (N, DP - D), 1)
    tail = jnp.where(cols == 0, 1.0, 0.0).astype(jnp.float32)
    hp_ref[...] = jnp.concatenate([h, tail], axis=1)
    as_ref[...] = a_s
    ad_ref[...] = a_d
    z = jnp.max(a_s) + a_d
    m_ref[...] = jnp.maximum(z, 0.2 * z)


def _prep2_body(s_ref, b_ref, w_ref, asv_ref, adv_ref,
                hp_ref, as_ref, ad_ref, m_ref):
    s = s_ref[0] + s_ref[1]
    feat = s[:, :D]
    den = jnp.sum(s[:, D:], axis=1)
    y = feat / (den[:, None] + 1e-16) + b_ref[...][None, :]
    y = jnp.maximum(y, 0.0)
    h = jnp.dot(y, w_ref[...], preferred_element_type=jnp.float32)
    a_s = jnp.dot(h, asv_ref[...], preferred_element_type=jnp.float32)
    a_d = jnp.dot(h, adv_ref[...], preferred_element_type=jnp.float32)
    cols = lax.broadcasted_iota(jnp.int32, (N, DP - D), 1)
    tail = jnp.where(cols == 0, 1.0, 0.0).astype(jnp.float32)
    hp_ref[...] = jnp.concatenate([h, tail], axis=1)
    as_ref[...] = a_s
    ad_ref[...] = a_d
    z = jnp.max(a_s) + a_d
    m_ref[...] = jnp.maximum(z, 0.2 * z)


def _final_body(s_ref, b_ref, out_ref):
    s = s_ref[0] + s_ref[1]
    feat = s[:, :D]
    den = jnp.sum(s[:, D:], axis=1)
    out_ref[...] = feat / (den[:, None] + 1e-16) + b_ref[...][None, :]


_node_out = [
    jax.ShapeDtypeStruct((N, DP), jnp.float32),
    jax.ShapeDtypeStruct((N,), jnp.float32),
    jax.ShapeDtypeStruct((N,), jnp.float32),
    jax.ShapeDtypeStruct((N,), jnp.float32),
]

_prep1 = pl.pallas_call(_prep1_body, out_shape=_node_out)
_prep2 = pl.pallas_call(_prep2_body, out_shape=_node_out)
_final = pl.pallas_call(
    _final_body, out_shape=jax.ShapeDtypeStruct((N, D), jnp.float32))


def _edge_body(hp, srcw, dstw, asrc, adst, mtab, out,
               s_acc, as_l, ad_l, m_l, src_l, dst_l, w_l, rows, zbuf):
    c_id = lax.axis_index("c")
    s_id = lax.axis_index("s")
    wid = s_id * NC + c_id
    base = s_id * NPT

    pltpu.sync_copy(asrc, as_l)
    pltpu.sync_copy(adst, ad_l)
    pltpu.sync_copy(mtab, m_l)
    pltpu.sync_copy(srcw.at[wid], src_l)
    pltpu.sync_copy(dstw.at[wid], dst_l)

    def zinit(i, carry):
        for k in range(DP // LANES):
            zbuf[i, pl.ds(k * LANES, LANES)] = jnp.zeros((LANES,), jnp.float32)
        return carry
    lax.fori_loop(0, 125, zinit, 0)
    for q in range(5):
        pltpu.sync_copy(zbuf, s_acc.at[pl.ds(base + 125 * q, 125)])
    plsc.subcore_barrier()

    def chunk(c, carry):
        pltpu.sync_copy(hp.at[src_l.at[c]], rows)

        for g in range(CH // LANES):
            sl = pl.ds(g * LANES, LANES)
            si = src_l[c, sl]
            di = dst_l[c, sl]
            asv = plsc.load_gather(as_l, [si])
            adv = plsc.load_gather(ad_l, [di])
            mv = plsc.load_gather(m_l, [di])
            t = asv + adv
            e = jnp.maximum(t, 0.2 * t)
            w_l[sl] = jnp.exp(e - mv)

        def scale(i, carry2):
            wv = plsc.load_gather(w_l, [jnp.full((LANES,), i, jnp.int32)])
            for k in range(DP // LANES):
                ksl = pl.ds(k * LANES, LANES)
                rows[i, ksl] = rows[i, ksl] * wv
            return carry2
        lax.fori_loop(0, CH, scale, 0)

        pltpu.sync_copy(rows, s_acc.at[dst_l.at[c]], add=True)
        return carry
    lax.fori_loop(0, NCHUNKS, chunk, 0)

    plsc.subcore_barrier()
    pltpu.sync_copy(s_acc.at[pl.ds(base, NPT)],
                    out.at[c_id, pl.ds(base, NPT)])


_edge = pl.kernel(
    _edge_body,
    out_type=jax.ShapeDtypeStruct((NC, N, DP), jnp.float32),
    mesh=plsc.VectorSubcoreMesh(core_axis_name="c", subcore_axis_name="s"),
    scratch_types=[
        pltpu.VMEM_SHARED((N, DP), jnp.float32),
        pltpu.VMEM((N,), jnp.float32),
        pltpu.VMEM((N,), jnp.float32),
        pltpu.VMEM((N,), jnp.float32),
        pltpu.VMEM((NCHUNKS, CH), jnp.int32),
        pltpu.VMEM((NCHUNKS, CH), jnp.int32),
        pltpu.VMEM((CH,), jnp.float32),
        pltpu.VMEM((CH, DP), jnp.float32),
        pltpu.VMEM((125, DP), jnp.float32),
    ],
)


@jax.jit
def kernel(x, edge_index, W1, att_src1, att_dst1, b1,
           W2, att_src2, att_dst2, b2):
    src3 = edge_index[0].reshape(NW, NCHUNKS, CH)
    dst3 = edge_index[1].reshape(NW, NCHUNKS, CH)

    hp1, as1, ad1, m1 = _prep1(x, W1, att_src1.reshape(D), att_dst1.reshape(D))
    part1 = _edge(hp1, src3, dst3, as1, ad1, m1)
    hp2, as2, ad2, m2 = _prep2(part1, b1, W2,
                               att_src2.reshape(D), att_dst2.reshape(D))
    part2 = _edge(hp2, src3, dst3, as2, ad2, m2)
    return _final(part2, b2)


# speedup vs baseline: 23.4078x; 23.4078x over previous
import functools

import jax
import jax.numpy as jnp
from jax import lax
from jax.experimental import pallas as pl
from jax.experimental.pallas import tpu as pltpu
from jax.experimental.pallas import tpu_sc as plsc

N = 10000
E = 320000
D = 128
DP = 144
NC = 2
NS = 16
NW = NC * NS
EPT = E // NW
CH = 80
GRP = 5
NGRP = EPT // (CH * GRP)
NPT = N // NS
LANES = 16


def _node_prep(h, asv, adv):
    a_s = jnp.dot(h, asv, preferred_element_type=jnp.float32)
    a_d = jnp.dot(h, adv, preferred_element_type=jnp.float32)
    cols = lax.broadcasted_iota(jnp.int32, (N, DP - D), 1)
    tail = jnp.where(cols == 0, 1.0, 0.0).astype(jnp.float32)
    hp = jnp.concatenate([h, tail], axis=1)
    gmax = jnp.full((D,), jnp.max(a_s), dtype=jnp.float32)
    return hp, a_s, a_d, gmax


def _prep1_body(x_ref, w_ref, asv_ref, adv_ref,
                hp_ref, as_ref, ad_ref, gm_ref):
    h = jnp.dot(x_ref[...], w_ref[...], preferred_element_type=jnp.float32)
    hp_ref[...], as_ref[...], ad_ref[...], gm_ref[...] = _node_prep(
        h, asv_ref[...], adv_ref[...])


def _prep2_body(s_ref, b_ref, w_ref, asv_ref, adv_ref,
                hp_ref, as_ref, ad_ref, gm_ref):
    s = s_ref[0] + s_ref[1]
    feat = s[:, :D]
    den = jnp.sum(s[:, D:], axis=1)
    y = feat / (den[:, None] + 1e-16) + b_ref[...][None, :]
    y = jnp.maximum(y, 0.0)
    h = jnp.dot(y, w_ref[...], preferred_element_type=jnp.float32)
    hp_ref[...], as_ref[...], ad_ref[...], gm_ref[...] = _node_prep(
        h, asv_ref[...], adv_ref[...])


def _final_body(s_ref, b_ref, out_ref):
    s = s_ref[0] + s_ref[1]
    feat = s[:, :D]
    den = jnp.sum(s[:, D:], axis=1)
    out_ref[...] = feat / (den[:, None] + 1e-16) + b_ref[...][None, :]


_node_out = [
    jax.ShapeDtypeStruct((N, DP), jnp.float32),
    jax.ShapeDtypeStruct((N,), jnp.float32),
    jax.ShapeDtypeStruct((N,), jnp.float32),
    jax.ShapeDtypeStruct((D,), jnp.float32),
]

_prep1 = pl.pallas_call(_prep1_body, out_shape=_node_out)
_prep2 = pl.pallas_call(_prep2_body, out_shape=_node_out)
_final = pl.pallas_call(
    _final_body, out_shape=jax.ShapeDtypeStruct((N, D), jnp.float32))


def _edge_body(hp, srcw, dstw, asrc, adst, gmax, out,
               s_acc, as_l, ad_l, gm_l, src_g, dst_g, w_l, rows):
    c_id = lax.axis_index("c")
    s_id = lax.axis_index("s")
    wid = s_id * NC + c_id
    base = s_id * NPT

    pltpu.sync_copy(asrc, as_l)
    pltpu.sync_copy(adst, ad_l)
    pltpu.sync_copy(gmax.at[pl.ds(0, LANES)], gm_l)

    def zinit(i, carry):
        for k in range(DP // LANES):
            rows[i, pl.ds(k * LANES, LANES)] = jnp.zeros((LANES,), jnp.float32)
        return carry
    lax.fori_loop(0, CH, zinit, 0)
    for q in range(7):
        pltpu.sync_copy(rows, s_acc.at[pl.ds(base + CH * q, CH)])
    pltpu.sync_copy(rows.at[pl.ds(0, NPT - 7 * CH)],
                    s_acc.at[pl.ds(base + 7 * CH, NPT - 7 * CH)])
    plsc.subcore_barrier()

    gv = gm_l[...]

    def group(g, carry):
        pltpu.sync_copy(srcw.at[wid, pl.ds(g * GRP, GRP)], src_g)
        pltpu.sync_copy(dstw.at[wid, pl.ds(g * GRP, GRP)], dst_g)
        for k in range(GRP):
            pltpu.sync_copy(hp.at[src_g.at[k]], rows)

            for gg in range(CH // LANES):
                sl = pl.ds(gg * LANES, LANES)
                asv = plsc.load_gather(as_l, [src_g[k, sl]])
                adv = plsc.load_gather(ad_l, [dst_g[k, sl]])
                t = asv + adv
                e = jnp.maximum(t, 0.2 * t)
                z = gv + adv
                mv = jnp.maximum(z, 0.2 * z)
                w_l[sl] = jnp.exp(e - mv)

            def scale(i, carry2):
                wv = plsc.load_gather(w_l, [jnp.full((LANES,), i, jnp.int32)])
                for kk in range(DP // LANES):
                    ksl = pl.ds(kk * LANES, LANES)
                    rows[i, ksl] = rows[i, ksl] * wv
                return carry2
            lax.fori_loop(0, CH, scale, 0)

            pltpu.sync_copy(rows, s_acc.at[dst_g.at[k]], add=True)
        return carry
    lax.fori_loop(0, NGRP, group, 0)

    plsc.subcore_barrier()
    pltpu.sync_copy(s_acc.at[pl.ds(base, NPT)],
                    out.at[c_id, pl.ds(base, NPT)])


_edge = pl.kernel(
    _edge_body,
    out_type=jax.ShapeDtypeStruct((NC, N, DP), jnp.float32),
    mesh=plsc.VectorSubcoreMesh(core_axis_name="c", subcore_axis_name="s",
                                num_cores=NC, num_subcores=NS),
    compiler_params=pltpu.CompilerParams(use_tc_tiling_on_sc=False,
                                         needs_layout_passes=False),
    scratch_types=[
        pltpu.VMEM_SHARED((N, DP), jnp.float32),
        pltpu.VMEM((N,), jnp.float32),
        pltpu.VMEM((N,), jnp.float32),
        pltpu.VMEM((LANES,), jnp.float32),
        pltpu.VMEM((GRP, CH), jnp.int32),
        pltpu.VMEM((GRP, CH), jnp.int32),
        pltpu.VMEM((CH,), jnp.float32),
        pltpu.VMEM((CH, DP), jnp.float32),
    ],
)


@jax.jit
def kernel(x, edge_index, W1, att_src1, att_dst1, b1,
           W2, att_src2, att_dst2, b2):
    src3 = edge_index[0].reshape(NW, EPT // CH, CH)
    dst3 = edge_index[1].reshape(NW, EPT // CH, CH)

    hp1, as1, ad1, gm1 = _prep1(x, W1, att_src1.reshape(D),
                                att_dst1.reshape(D))
    part1 = _edge(hp1, src3, dst3, as1, ad1, gm1)
    hp2, as2, ad2, gm2 = _prep2(part1, b1, W2,
                                att_src2.reshape(D), att_dst2.reshape(D))
    part2 = _edge(hp2, src3, dst3, as2, ad2, gm2)
    return _final(part2, b2)


# speedup vs baseline: 25.0138x; 1.0686x over previous
import functools

import jax
import jax.numpy as jnp
from jax import lax
from jax.experimental import pallas as pl
from jax.experimental.pallas import tpu as pltpu
from jax.experimental.pallas import tpu_sc as plsc

N = 10000
E = 320000
D = 128
DP = 144
NC = 2
NS = 16
NW = NC * NS
EPT = E // NW
CH = 64
NCH = 158
EPTP = NCH * CH
NPT = N // NS
LANES = 16


def _node_prep(h, asv, adv, hp_ref, ad_ref, gm_ref):
    a_s = jnp.dot(h, asv, preferred_element_type=jnp.float32)
    a_d = jnp.dot(h, adv, preferred_element_type=jnp.float32)
    cols = lax.broadcasted_iota(jnp.int32, (N, DP - D), 1)
    tail = jnp.where(cols == 0, 1.0,
                     jnp.where(cols == 1, a_s[:, None], 0.0))
    hp_ref[:, :D] = h
    hp_ref[:, D:] = tail.astype(jnp.float32)
    ad_ref[...] = a_d
    gm_ref[...] = jnp.full((D,), jnp.max(a_s), dtype=jnp.float32)


def _prep1_body(x_ref, w_ref, asv_ref, adv_ref, hp_ref, ad_ref, gm_ref):
    h = jnp.dot(x_ref[...], w_ref[...], preferred_element_type=jnp.float32)
    _node_prep(h, asv_ref[...], adv_ref[...], hp_ref, ad_ref, gm_ref)


def _combine(s):
    feat = s[:, :D]
    cols = lax.broadcasted_iota(jnp.int32, (N, DP - D), 1)
    den = jnp.sum(jnp.where(cols == 0, s[:, D:], 0.0), axis=1)
    return feat, den


def _prep2_body(s_ref, b_ref, w_ref, asv_ref, adv_ref,
                hp_ref, ad_ref, gm_ref):
    feat, den = _combine(s_ref[0] + s_ref[1])
    y = feat / (den[:, None] + 1e-16) + b_ref[...][None, :]
    y = jnp.maximum(y, 0.0)
    h = jnp.dot(y, w_ref[...], preferred_element_type=jnp.float32)
    _node_prep(h, asv_ref[...], adv_ref[...], hp_ref, ad_ref, gm_ref)


def _final_body(s_ref, b_ref, out_ref):
    feat, den = _combine(s_ref[0] + s_ref[1])
    out_ref[...] = feat / (den[:, None] + 1e-16) + b_ref[...][None, :]


_node_out = [
    jax.ShapeDtypeStruct((N, DP), jnp.float32),
    jax.ShapeDtypeStruct((N,), jnp.float32),
    jax.ShapeDtypeStruct((D,), jnp.float32),
]

_prep1 = pl.pallas_call(_prep1_body, out_shape=_node_out)
_prep2 = pl.pallas_call(_prep2_body, out_shape=_node_out)
_final = pl.pallas_call(
    _final_body, out_shape=jax.ShapeDtypeStruct((N, D), jnp.float32))


def _edge_body(hp, pk, adst, gmax, out,
               s_acc, ad_l, gm_l, pk_l,
               src0, src1, dst0, dst1, w_l, rows0, rows1,
               gsem, ssem):
    c_id = lax.axis_index("c")
    s_id = lax.axis_index("s")
    wid = s_id * NC + c_id
    base = s_id * NPT

    pltpu.sync_copy(adst, ad_l)
    pltpu.sync_copy(gmax.at[pl.ds(0, LANES)], gm_l)
    pltpu.sync_copy(pk.at[wid], pk_l)

    srcb = (src0, src1)
    dstb = (dst0, dst1)
    rowsb = (rows0, rows1)

    def zinit(i, carry):
        for k in range(DP // LANES):
            rows0[i, pl.ds(k * LANES, LANES)] = jnp.zeros((LANES,),
                                                          jnp.float32)
        return carry
    lax.fori_loop(0, CH, zinit, 0)
    for q in range(9):
        pltpu.sync_copy(rows0, s_acc.at[pl.ds(base + CH * q, CH)])
    pltpu.sync_copy(rows0.at[pl.ds(0, NPT - 9 * CH)],
                    s_acc.at[pl.ds(base + 9 * CH, NPT - 9 * CH)])
    plsc.subcore_barrier()

    gv = gm_l[...]

    def unpack(c, b):
        for g in range(CH // LANES):
            sl = pl.ds(g * LANES, LANES)
            p = pk_l[pl.ds(c * CH + g * LANES, LANES)]
            srcb[b][sl] = p & 0xFFFF
            dstb[b][sl] = lax.shift_right_logical(p, 16)

    def g_desc(b):
        return pltpu.make_async_copy(hp.at[srcb[b]], rowsb[b], gsem)

    def s_desc(b):
        return pltpu.make_async_copy(rowsb[b], s_acc.at[dstb[b]], ssem)

    def compute(c, b):
        rows = rowsb[b]
        for g in range(CH // LANES):
            sl = pl.ds(g * LANES, LANES)
            lane = lax.iota(jnp.int32, LANES)
            asv = plsc.load_gather(
                rows, [lane + g * LANES,
                       jnp.full((LANES,), D + 1, jnp.int32)])
            adv = plsc.load_gather(ad_l, [dstb[b][sl]])
            t = asv + adv
            e = jnp.maximum(t, 0.2 * t)
            z = gv + adv
            mv = jnp.maximum(z, 0.2 * z)
            w = jnp.exp(e - mv)
            eid = c * CH + g * LANES + lane
            w_l[sl] = jnp.where(eid < EPT, w, 0.0)

        def scale(i, carry):
            wv = plsc.load_gather(w_l, [jnp.full((LANES,), i, jnp.int32)])
            for k in range(DP // LANES):
                ksl = pl.ds(k * LANES, LANES)
                rows[i, ksl] = rows[i, ksl] * wv
            return carry
        lax.fori_loop(0, CH, scale, 0)

    unpack(0, 0)
    g_desc(0).start()

    def pair(i, carry):
        for b in range(2):
            c = 2 * i + b
            g_desc(b).wait()

            @pl.when(c > 0)
            def _():
                s_desc(1 - b).wait()

            @pl.when(c + 1 < NCH)
            def _():
                unpack(c + 1, 1 - b)
                g_desc(1 - b).start()
            compute(c, b)
            s_desc(b).start(add=True)
        return carry
    lax.fori_loop(0, NCH // 2, pair, 0)

    s_desc(1).wait()

    plsc.subcore_barrier()
    pltpu.sync_copy(s_acc.at[pl.ds(base, NPT)],
                    out.at[c_id, pl.ds(base, NPT)])


_edge = pl.kernel(
    _edge_body,
    out_type=jax.ShapeDtypeStruct((NC, N, DP), jnp.float32),
    mesh=plsc.VectorSubcoreMesh(core_axis_name="c", subcore_axis_name="s",
                                num_cores=NC, num_subcores=NS),
    compiler_params=pltpu.CompilerParams(use_tc_tiling_on_sc=False,
                                         needs_layout_passes=False),
    scratch_types=[
        pltpu.VMEM_SHARED((N, DP), jnp.float32),
        pltpu.VMEM((N,), jnp.float32),
        pltpu.VMEM((LANES,), jnp.float32),
        pltpu.VMEM((EPTP,), jnp.int32),
        pltpu.VMEM((CH,), jnp.int32),
        pltpu.VMEM((CH,), jnp.int32),
        pltpu.VMEM((CH,), jnp.int32),
        pltpu.VMEM((CH,), jnp.int32),
        pltpu.VMEM((CH,), jnp.float32),
        pltpu.VMEM((CH, DP), jnp.float32),
        pltpu.VMEM((CH, DP), jnp.float32),
        pltpu.SemaphoreType.DMA,
        pltpu.SemaphoreType.DMA,
    ],
)


@jax.jit
def kernel(x, edge_index, W1, att_src1, att_dst1, b1,
           W2, att_src2, att_dst2, b2):
    packed = jnp.left_shift(edge_index[1], 16) | edge_index[0]
    packed = packed.reshape(NW, EPT)
    packed = jnp.pad(packed, ((0, 0), (0, EPTP - EPT)))

    hp1, ad1, gm1 = _prep1(x, W1, att_src1.reshape(D), att_dst1.reshape(D))
    part1 = _edge(hp1, packed, ad1, gm1)
    hp2, ad2, gm2 = _prep2(part1, b1, W2,
                           att_src2.reshape(D), att_dst2.reshape(D))
    part2 = _edge(hp2, packed, ad2, gm2)
    return _final(part2, b2)


# speedup vs baseline: 26.2643x; 1.0500x over previous
import functools

import jax
import jax.numpy as jnp
from jax import lax
from jax.experimental import pallas as pl
from jax.experimental.pallas import tpu as pltpu
from jax.experimental.pallas import tpu_sc as plsc

N = 10000
E = 320000
D = 128
DP = 144
NC = 2
NS = 16
NW = NC * NS
EPT = E // NW
CH = 64
NCH = 158
EPTP = NCH * CH
NPT = N // NS
LANES = 16


def _node_prep(h, asv, adv, hp_ref, ad_ref, gm_ref):
    a_s = jnp.dot(h, asv, preferred_element_type=jnp.float32)
    a_d = jnp.dot(h, adv, preferred_element_type=jnp.float32)
    cols = lax.broadcasted_iota(jnp.int32, (N, DP - D), 1)
    tail = jnp.where(cols == 0, 1.0,
                     jnp.where(cols == 1, a_s[:, None], 0.0))
    hp_ref[:, :D] = h
    hp_ref[:, D:] = tail.astype(jnp.float32)
    ad_ref[...] = a_d
    gm_ref[...] = jnp.full((D,), jnp.max(a_s), dtype=jnp.float32)


def _prep1_body(x_ref, w_ref, asv_ref, adv_ref, hp_ref, ad_ref, gm_ref):
    h = jnp.dot(x_ref[...], w_ref[...], preferred_element_type=jnp.float32)
    _node_prep(h, asv_ref[...], adv_ref[...], hp_ref, ad_ref, gm_ref)


def _combine(s):
    feat = s[:, :D]
    cols = lax.broadcasted_iota(jnp.int32, (N, DP - D), 1)
    den = jnp.sum(jnp.where(cols == 0, s[:, D:], 0.0), axis=1)
    return feat, den


def _prep2_body(s_ref, b_ref, w_ref, asv_ref, adv_ref,
                hp_ref, ad_ref, gm_ref):
    feat, den = _combine(s_ref[0] + s_ref[1])
    y = feat / (den[:, None] + 1e-16) + b_ref[...][None, :]
    y = jnp.maximum(y, 0.0)
    h = jnp.dot(y, w_ref[...], preferred_element_type=jnp.float32)
    _node_prep(h, asv_ref[...], adv_ref[...], hp_ref, ad_ref, gm_ref)


def _final_body(s_ref, b_ref, out_ref):
    feat, den = _combine(s_ref[0] + s_ref[1])
    out_ref[...] = feat / (den[:, None] + 1e-16) + b_ref[...][None, :]


_node_out = [
    jax.ShapeDtypeStruct((N, DP), jnp.float32),
    jax.ShapeDtypeStruct((N,), jnp.float32),
    jax.ShapeDtypeStruct((D,), jnp.float32),
]

_prep1 = pl.pallas_call(_prep1_body, out_shape=_node_out)
_prep2 = pl.pallas_call(_prep2_body, out_shape=_node_out)
_final = pl.pallas_call(
    _final_body, out_shape=jax.ShapeDtypeStruct((N, D), jnp.float32))


def _edge_body(hp, pk, adst, gmax, out,
               s_acc, ad_l, gm_l, pk_l,
               src0, src1, dst0, dst1, w_l, rows0, rows1,
               gsem, ssem):
    c_id = lax.axis_index("c")
    s_id = lax.axis_index("s")
    wid = s_id * NC + c_id
    base = s_id * NPT

    pltpu.sync_copy(adst, ad_l)
    pltpu.sync_copy(gmax.at[pl.ds(0, LANES)], gm_l)
    pltpu.sync_copy(pk.at[wid], pk_l)

    srcb = (src0, src1)
    dstb = (dst0, dst1)
    rowsb = (rows0, rows1)

    @plsc.parallel_loop(0, CH, 1, unroll=4)
    def zinit(i):
        for k in range(DP // LANES):
            rows0[i, pl.ds(k * LANES, LANES)] = jnp.zeros((LANES,),
                                                          jnp.float32)
    for q in range(9):
        pltpu.sync_copy(rows0, s_acc.at[pl.ds(base + CH * q, CH)])
    pltpu.sync_copy(rows0.at[pl.ds(0, NPT - 9 * CH)],
                    s_acc.at[pl.ds(base + 9 * CH, NPT - 9 * CH)])
    plsc.subcore_barrier()

    gv = gm_l[...]

    def unpack(c, b):
        for g in range(CH // LANES):
            sl = pl.ds(g * LANES, LANES)
            p = pk_l[pl.ds(c * CH + g * LANES, LANES)]
            srcb[b][sl] = p & 0xFFFF
            dstb[b][sl] = lax.shift_right_logical(p, 16)

    def g_desc(b):
        return pltpu.make_async_copy(hp.at[srcb[b]], rowsb[b], gsem)

    def s_desc(b):
        return pltpu.make_async_copy(rowsb[b], s_acc.at[dstb[b]], ssem)

    def compute(c, b):
        rows = rowsb[b]
        for g in range(CH // LANES):
            sl = pl.ds(g * LANES, LANES)
            lane = lax.iota(jnp.int32, LANES)
            asv = plsc.load_gather(
                rows, [lane + g * LANES,
                       jnp.full((LANES,), D + 1, jnp.int32)])
            adv = plsc.load_gather(ad_l, [dstb[b][sl]])
            t = asv + adv
            e = jnp.maximum(t, 0.2 * t)
            z = gv + adv
            mv = jnp.maximum(z, 0.2 * z)
            w = jnp.exp(e - mv)
            eid = c * CH + g * LANES + lane
            w_l[sl] = jnp.where(eid < EPT, w, 0.0)

        @plsc.parallel_loop(0, CH, 1, unroll=4)
        def scale(i):
            wv = plsc.load_gather(w_l, [jnp.full((LANES,), i, jnp.int32)])
            for k in range(DP // LANES):
                ksl = pl.ds(k * LANES, LANES)
                rows[i, ksl] = rows[i, ksl] * wv

    unpack(0, 0)
    g_desc(0).start()

    def pair(i, carry):
        for b in range(2):
            c = 2 * i + b
            g_desc(b).wait()

            @pl.when(c > 0)
            def _():
                s_desc(1 - b).wait()

            @pl.when(c + 1 < NCH)
            def _():
                unpack(c + 1, 1 - b)
                g_desc(1 - b).start()
            compute(c, b)
            s_desc(b).start(add=True)
        return carry
    lax.fori_loop(0, NCH // 2, pair, 0)

    s_desc(1).wait()

    plsc.subcore_barrier()
    pltpu.sync_copy(s_acc.at[pl.ds(base, NPT)],
                    out.at[c_id, pl.ds(base, NPT)])


_edge = pl.kernel(
    _edge_body,
    out_type=jax.ShapeDtypeStruct((NC, N, DP), jnp.float32),
    mesh=plsc.VectorSubcoreMesh(core_axis_name="c", subcore_axis_name="s",
                                num_cores=NC, num_subcores=NS),
    compiler_params=pltpu.CompilerParams(use_tc_tiling_on_sc=False,
                                         needs_layout_passes=False),
    scratch_types=[
        pltpu.VMEM_SHARED((N, DP), jnp.float32),
        pltpu.VMEM((N,), jnp.float32),
        pltpu.VMEM((LANES,), jnp.float32),
        pltpu.VMEM((EPTP,), jnp.int32),
        pltpu.VMEM((CH,), jnp.int32),
        pltpu.VMEM((CH,), jnp.int32),
        pltpu.VMEM((CH,), jnp.int32),
        pltpu.VMEM((CH,), jnp.int32),
        pltpu.VMEM((CH,), jnp.float32),
        pltpu.VMEM((CH, DP), jnp.float32),
        pltpu.VMEM((CH, DP), jnp.float32),
        pltpu.SemaphoreType.DMA,
        pltpu.SemaphoreType.DMA,
    ],
)


@jax.jit
def kernel(x, edge_index, W1, att_src1, att_dst1, b1,
           W2, att_src2, att_dst2, b2):
    packed = jnp.left_shift(edge_index[1], 16) | edge_index[0]
    packed = packed.reshape(NW, EPT)
    packed = jnp.pad(packed, ((0, 0), (0, EPTP - EPT)))

    hp1, ad1, gm1 = _prep1(x, W1, att_src1.reshape(D), att_dst1.reshape(D))
    part1 = _edge(hp1, packed, ad1, gm1)
    hp2, ad2, gm2 = _prep2(part1, b1, W2,
                           att_src2.reshape(D), att_dst2.reshape(D))
    part2 = _edge(hp2, packed, ad2, gm2)
    return _final(part2, b2)


# speedup vs baseline: 26.3586x; 1.0036x over previous
import functools

import jax
import jax.numpy as jnp
from jax import lax
from jax.experimental import pallas as pl
from jax.experimental.pallas import tpu as pltpu
from jax.experimental.pallas import tpu_sc as plsc

N = 10000
E = 320000
D = 128
DP = 144
PW = 80
NC = 2
NS = 16
NW = NC * NS
EPT = E // NW
CH = 32
NCH = 316
EPTP = NCH * CH
NPT = N // NS
LANES = 16


def _node_prep(h, asv, adv, hp_ref, ad_ref, gm_ref):
    a_s = jnp.dot(h, asv, preferred_element_type=jnp.float32)
    a_d = jnp.dot(h, adv, preferred_element_type=jnp.float32)
    cols = lax.broadcasted_iota(jnp.int32, (N, 16), 1)
    tail = jnp.where(cols == 0, 1.0,
                     jnp.where(cols == 1, a_s[:, None], 0.0))
    lo = h[:, :PW]
    hi = jnp.concatenate([h[:, PW:], tail.astype(jnp.float32),
                          jnp.zeros((N, 16), jnp.float32)], axis=1)
    lo16 = lax.bitcast_convert_type(lo.astype(jnp.bfloat16), jnp.uint16)
    hi16 = lax.bitcast_convert_type(hi.astype(jnp.bfloat16), jnp.uint16)
    packed = lo16.astype(jnp.uint32) | (hi16.astype(jnp.uint32) << 16)
    hp_ref[...] = lax.bitcast_convert_type(packed, jnp.int32)
    ad_ref[...] = a_d
    gm_ref[...] = jnp.full((D,), jnp.max(a_s), dtype=jnp.float32)


def _prep1_body(x_ref, w_ref, asv_ref, adv_ref, hp_ref, ad_ref, gm_ref):
    h = jnp.dot(x_ref[...], w_ref[...], preferred_element_type=jnp.float32)
    _node_prep(h, asv_ref[...], adv_ref[...], hp_ref, ad_ref, gm_ref)


def _combine(s):
    feat = s[:, :D]
    cols = lax.broadcasted_iota(jnp.int32, (N, DP - D), 1)
    den = jnp.sum(jnp.where(cols == 0, s[:, D:], 0.0), axis=1)
    return feat, den


def _prep2_body(s_ref, b_ref, w_ref, asv_ref, adv_ref,
                hp_ref, ad_ref, gm_ref):
    feat, den = _combine(s_ref[0] + s_ref[1])
    y = feat / (den[:, None] + 1e-16) + b_ref[...][None, :]
    y = jnp.maximum(y, 0.0)
    h = jnp.dot(y, w_ref[...], preferred_element_type=jnp.float32)
    _node_prep(h, asv_ref[...], adv_ref[...], hp_ref, ad_ref, gm_ref)


def _final_body(s_ref, b_ref, out_ref):
    feat, den = _combine(s_ref[0] + s_ref[1])
    out_ref[...] = feat / (den[:, None] + 1e-16) + b_ref[...][None, :]


_node_out = [
    jax.ShapeDtypeStruct((N, PW), jnp.int32),
    jax.ShapeDtypeStruct((N,), jnp.float32),
    jax.ShapeDtypeStruct((D,), jnp.float32),
]

_tc_params = pltpu.CompilerParams(vmem_limit_bytes=100 * 1024 * 1024)
_prep1 = pl.pallas_call(_prep1_body, out_shape=_node_out,
                        compiler_params=_tc_params)
_prep2 = pl.pallas_call(_prep2_body, out_shape=_node_out,
                        compiler_params=_tc_params)
_final = pl.pallas_call(
    _final_body, out_shape=jax.ShapeDtypeStruct((N, D), jnp.float32),
    compiler_params=_tc_params)


def _edge_body(hp, pk, adst, gmax, out,
               s_acc, ad_l, gm_l, pk_l,
               src0, src1, dst0, dst1,
               w_l, rows0, rows1, sbuf0, sbuf1,
               gsem, ssem):
    c_id = lax.axis_index("c")
    s_id = lax.axis_index("s")
    wid = s_id * NC + c_id
    base = s_id * NPT

    pltpu.sync_copy(adst, ad_l)
    pltpu.sync_copy(gmax.at[pl.ds(0, LANES)], gm_l)
    pltpu.sync_copy(pk.at[wid], pk_l)

    srcb = (src0, src1)
    dstb = (dst0, dst1)
    rowsb = (rows0, rows1)
    sbufb = (sbuf0, sbuf1)

    @plsc.parallel_loop(0, CH, 1, unroll=4)
    def zinit(i):
        for k in range(DP // LANES):
            sbuf0[i, pl.ds(k * LANES, LANES)] = jnp.zeros((LANES,),
                                                          jnp.float32)
    for q in range(NPT // CH):
        pltpu.sync_copy(sbuf0, s_acc.at[pl.ds(base + CH * q, CH)])
    rem = NPT - (NPT // CH) * CH
    pltpu.sync_copy(sbuf0.at[pl.ds(0, rem)],
                    s_acc.at[pl.ds(base + NPT - rem, rem)])
    plsc.subcore_barrier()

    gv = gm_l[...]

    def unpack(c, b):
        for g in range(CH // LANES):
            sl = pl.ds(g * LANES, LANES)
            p = pk_l[pl.ds(c * CH + g * LANES, LANES)]
            srcb[b][sl] = p & 0xFFFF
            dstb[b][sl] = lax.shift_right_logical(p, 16)

    def g_desc(b):
        return pltpu.make_async_copy(hp.at[srcb[b]], rowsb[b], gsem)

    def s_desc(b):
        return pltpu.make_async_copy(sbufb[b], s_acc.at[dstb[b]], ssem)

    def hi_f32(word):
        mask = jnp.full((LANES,), -65536, jnp.int32)
        return plsc.bitcast(jnp.bitwise_and(word, mask), jnp.float32)

    def lo_f32(word):
        sh = jnp.full((LANES,), 16, jnp.int32)
        return plsc.bitcast(jnp.left_shift(word, sh), jnp.float32)

    def compute(c, b):
        rows = rowsb[b]
        sbuf = sbufb[b]
        for g in range(CH // LANES):
            sl = pl.ds(g * LANES, LANES)
            lane = lax.iota(jnp.int32, LANES)
            asw = plsc.load_gather(
                rows, [lane + g * LANES,
                       jnp.full((LANES,), 49, jnp.int32)])
            asv = hi_f32(asw)
            adv = plsc.load_gather(ad_l, [dstb[b][sl]])
            t = asv + adv
            e = jnp.maximum(t, 0.2 * t)
            z = gv + adv
            mv = jnp.maximum(z, 0.2 * z)
            w = jnp.exp(e - mv)
            eid = c * CH + g * LANES + lane
            w_l[sl] = jnp.where(eid < EPT, w, 0.0)

        @plsc.parallel_loop(0, CH, 1, unroll=4)
        def scale(i):
            wv = plsc.load_gather(w_l, [jnp.full((LANES,), i, jnp.int32)])
            for g5 in range(5):
                ksl = pl.ds(g5 * LANES, LANES)
                word = rows[i, ksl]
                sbuf[i, ksl] = lo_f32(word) * wv
                if g5 < 4:
                    hsl = pl.ds(PW + g5 * LANES, LANES)
                    sbuf[i, hsl] = hi_f32(word) * wv

    unpack(0, 0)
    g_desc(0).start()

    def pair(i, carry):
        for b in range(2):
            c = 2 * i + b
            g_desc(b).wait()

            @pl.when(c > 0)
            def _():
                s_desc(1 - b).wait()

            @pl.when(c + 1 < NCH)
            def _():
                unpack(c + 1, 1 - b)
                g_desc(1 - b).start()
            compute(c, b)
            s_desc(b).start(add=True)
        return carry
    lax.fori_loop(0, NCH // 2, pair, 0)

    s_desc(1).wait()

    plsc.subcore_barrier()
    pltpu.sync_copy(s_acc.at[pl.ds(base, NPT)],
                    out.at[c_id, pl.ds(base, NPT)])


_edge = pl.kernel(
    _edge_body,
    out_type=jax.ShapeDtypeStruct((NC, N, DP), jnp.float32),
    mesh=plsc.VectorSubcoreMesh(core_axis_name="c", subcore_axis_name="s",
                                num_cores=NC, num_subcores=NS),
    compiler_params=pltpu.CompilerParams(use_tc_tiling_on_sc=False,
                                         needs_layout_passes=False),
    scratch_types=[
        pltpu.VMEM_SHARED((N, DP), jnp.float32),
        pltpu.VMEM((N,), jnp.float32),
        pltpu.VMEM((LANES,), jnp.float32),
        pltpu.VMEM((EPTP,), jnp.int32),
        pltpu.VMEM((CH,), jnp.int32),
        pltpu.VMEM((CH,), jnp.int32),
        pltpu.VMEM((CH,), jnp.int32),
        pltpu.VMEM((CH,), jnp.int32),
        pltpu.VMEM((CH,), jnp.float32),
        pltpu.VMEM((CH, PW), jnp.int32),
        pltpu.VMEM((CH, PW), jnp.int32),
        pltpu.VMEM((CH, DP), jnp.float32),
        pltpu.VMEM((CH, DP), jnp.float32),
        pltpu.SemaphoreType.DMA,
        pltpu.SemaphoreType.DMA,
    ],
)


@jax.jit
def kernel(x, edge_index, W1, att_src1, att_dst1, b1,
           W2, att_src2, att_dst2, b2):
    packed = jnp.left_shift(edge_index[1], 16) | edge_index[0]
    packed = packed.reshape(NW, EPT)
    packed = jnp.pad(packed, ((0, 0), (0, EPTP - EPT)))

    hp1, ad1, gm1 = _prep1(x, W1, att_src1.reshape(D), att_dst1.reshape(D))
    part1 = _edge(hp1, packed, ad1, gm1)
    hp2, ad2, gm2 = _prep2(part1, b1, W2,
                           att_src2.reshape(D), att_dst2.reshape(D))
    part2 = _edge(hp2, packed, ad2, gm2)
    return _final(part2, b2)
